# baseline (speedup 1.0000x reference)
import functools

import jax
import jax.numpy as jnp
from jax import lax
from jax.experimental import pallas as pl
from jax.experimental.pallas import tpu as pltpu
from jax.experimental.pallas import tpu_sc as plsc

DIM = 64
EPS = 1e-5
NC = 2
NS = 16
L = 16
NW = NC * NS
CHUNK = 128


def _rsqrt(x):
    i = plsc.bitcast(x, jnp.int32)
    i = jnp.int32(0x5F3759DF) - lax.shift_right_arithmetic(i, 1)
    y = plsc.bitcast(i, jnp.float32)
    for _ in range(3):
        y = y * (1.5 - 0.5 * x * y * y)
    return y


def _emb_ln_body(idx_hbm, table_hbm, gamma_hbm, beta_hbm, out_hbm,
                 idx_v, buf, gam_v, bet_v, gsem, osem, per_w):
    wid = lax.axis_index("s") * NC + lax.axis_index("c")
    base = wid * per_w
    nchunk = per_w // CHUNK

    pltpu.sync_copy(gamma_hbm, gam_v)
    pltpu.sync_copy(beta_hbm, bet_v)
    pltpu.sync_copy(idx_hbm.at[pl.ds(base, per_w)], idx_v)

    lanes = lax.iota(jnp.int32, L)

    def chunk_body(c, _):
        idx_slice = idx_v.at[pl.ds(c * CHUNK, CHUNK)]
        pltpu.async_copy(table_hbm.at[idx_slice], buf, gsem).wait()

        def group_body(g, _):
            rows = lanes + g * L
            s = jnp.zeros((L,), jnp.float32)
            q = jnp.zeros((L,), jnp.float32)
            for d in range(DIM):
                cols = jnp.full((L,), d, jnp.int32)
                v = plsc.load_gather(buf, [rows, cols])
                s = s + v
                q = q + v * v
            mean = s * (1.0 / DIM)
            var = q * (1.0 / DIM) - mean * mean
            rstd = _rsqrt(var + EPS)
            shift = -mean * rstd
            for d in range(DIM):
                cols = jnp.full((L,), d, jnp.int32)
                v = plsc.load_gather(buf, [rows, cols])
                plsc.store_scatter(buf, [rows, cols], v * rstd + shift)
            for r in range(L):
                row = g * L + r
                for k in range(DIM // L):
                    gk = gam_v[pl.ds(k * L, L)]
                    bk = bet_v[pl.ds(k * L, L)]
                    n = buf[row, pl.ds(k * L, L)]
                    buf[row, pl.ds(k * L, L)] = n * gk + bk
            return _

        lax.fori_loop(0, CHUNK // L, group_body, None)
        pltpu.sync_copy(buf, out_hbm.at[pl.ds(base + c * CHUNK, CHUNK)])
        return _

    lax.fori_loop(0, nchunk, chunk_body, None)


def kernel(x, table, gamma, beta):
    b, h = x.shape
    n = b * h
    per_w = n // NW
    idx_flat = x.reshape(n).astype(jnp.int32)

    mesh = plsc.VectorSubcoreMesh(
        core_axis_name="c", subcore_axis_name="s",
        num_cores=NC, num_subcores=NS)
    body = functools.partial(_emb_ln_body, per_w=per_w)
    run = pl.kernel(
        body,
        out_type=jax.ShapeDtypeStruct((n, DIM), jnp.float32),
        mesh=mesh,
        compiler_params=pltpu.CompilerParams(
            needs_layout_passes=False, use_tc_tiling_on_sc=False),
        scratch_types=[
            pltpu.VMEM((per_w,), jnp.int32),
            pltpu.VMEM((CHUNK, DIM), jnp.float32),
            pltpu.VMEM((DIM,), jnp.float32),
            pltpu.VMEM((DIM,), jnp.float32),
            pltpu.SemaphoreType.DMA,
            pltpu.SemaphoreType.DMA,
        ],
    )
    out = run(idx_flat, table, gamma, beta)
    return out.reshape(b, h, DIM)


# speedup vs baseline: 1.0596x; 1.0596x over previous
import functools

import jax
import jax.numpy as jnp
from jax import lax
from jax.experimental import pallas as pl
from jax.experimental.pallas import tpu as pltpu
from jax.experimental.pallas import tpu_sc as plsc

DIM = 64
EPS = 1e-5
NC = 2
NS = 16
L = 16
NW = NC * NS
CHUNK = 128
NBUF = 4


def _rsqrt(x):
    i = plsc.bitcast(x, jnp.int32)
    i = jnp.int32(0x5F3759DF) - lax.shift_right_arithmetic(i, 1)
    y = plsc.bitcast(i, jnp.float32)
    for _ in range(3):
        y = y * (1.5 - 0.5 * x * y * y)
    return y


def _emb_ln_body(idx_hbm, table_hbm, gamma_hbm, beta_hbm, out_hbm,
                 idx_v, buf, gam_v, bet_v, gsems, osems, per_w):
    wid = lax.axis_index("s") * NC + lax.axis_index("c")
    base = wid * per_w
    nchunk = per_w // CHUNK

    pltpu.sync_copy(gamma_hbm, gam_v)
    pltpu.sync_copy(beta_hbm, bet_v)
    pltpu.sync_copy(idx_hbm.at[pl.ds(base, per_w)], idx_v)

    lanes = lax.iota(jnp.int32, L)

    def gather_src(c):
        return table_hbm.at[idx_v.at[pl.ds(c * CHUNK, CHUNK)]]

    def out_dst(c):
        return out_hbm.at[pl.ds(base + c * CHUNK, CHUNK)]

    def compute(s):
        bufs = buf.at[s]

        def group_body(g, _):
            rows = lanes + g * L
            su = jnp.zeros((L,), jnp.float32)
            q = jnp.zeros((L,), jnp.float32)
            for d in range(DIM):
                cols = jnp.full((L,), d, jnp.int32)
                v = plsc.load_gather(bufs, [rows, cols])
                su = su + v
                q = q + v * v
            mean = su * (1.0 / DIM)
            var = q * (1.0 / DIM) - mean * mean
            rstd = _rsqrt(var + EPS)
            shift = -mean * rstd
            for d in range(DIM):
                cols = jnp.full((L,), d, jnp.int32)
                v = plsc.load_gather(bufs, [rows, cols])
                plsc.store_scatter(bufs, [rows, cols], v * rstd + shift)
            for r in range(L):
                row = g * L + r
                for k in range(DIM // L):
                    gk = gam_v[pl.ds(k * L, L)]
                    bk = bet_v[pl.ds(k * L, L)]
                    n = buf[s, row, pl.ds(k * L, L)]
                    buf[s, row, pl.ds(k * L, L)] = n * gk + bk
            return _

        lax.fori_loop(0, CHUNK // L, group_body, None)

    pltpu.async_copy(gather_src(0), buf.at[0], gsems[0])

    def outer(c4, _):
        for b in range(NBUF):
            c = c4 * NBUF + b
            s_next = (b + 1) % NBUF

            @pl.when(c >= NBUF - 1)
            def _wait_out():
                co = c - (NBUF - 1)
                pltpu.make_async_copy(
                    buf.at[s_next], out_dst(co), osems[s_next]).wait()

            @pl.when(c < nchunk - 1)
            def _issue_gather():
                pltpu.async_copy(
                    gather_src(c + 1), buf.at[s_next], gsems[s_next])

            pltpu.make_async_copy(gather_src(c), buf.at[b], gsems[b]).wait()
            compute(b)
            pltpu.async_copy(buf.at[b], out_dst(c), osems[b])
        return _

    lax.fori_loop(0, nchunk // NBUF, outer, None)

    for b in range(1, NBUF):
        c = nchunk - NBUF + b
        pltpu.make_async_copy(buf.at[c % NBUF], out_dst(c),
                              osems[c % NBUF]).wait()


def kernel(x, table, gamma, beta):
    b, h = x.shape
    n = b * h
    per_w = n // NW
    idx_flat = x.reshape(n).astype(jnp.int32)

    mesh = plsc.VectorSubcoreMesh(
        core_axis_name="c", subcore_axis_name="s",
        num_cores=NC, num_subcores=NS)
    body = functools.partial(_emb_ln_body, per_w=per_w)
    run = pl.kernel(
        body,
        out_type=jax.ShapeDtypeStruct((n, DIM), jnp.float32),
        mesh=mesh,
        compiler_params=pltpu.CompilerParams(
            needs_layout_passes=False, use_tc_tiling_on_sc=False,
            disable_bounds_checks=True),
        scratch_types=[
            pltpu.VMEM((per_w,), jnp.int32),
            pltpu.VMEM((NBUF, CHUNK, DIM), jnp.float32),
            pltpu.VMEM((DIM,), jnp.float32),
            pltpu.VMEM((DIM,), jnp.float32),
            [pltpu.SemaphoreType.DMA] * NBUF,
            [pltpu.SemaphoreType.DMA] * NBUF,
        ],
    )
    out = run(idx_flat, table, gamma, beta)
    return out.reshape(b, h, DIM)
